# Initial kernel scaffold; baseline (speedup 1.0000x reference)
import jax
import jax.numpy as jnp
from jax.experimental import pallas as pl


def kernel(x, W, b):
    raise NotImplementedError("write your pallas kernel here")


# speedup vs baseline: 789.6341x; 789.6341x over previous
import functools

import jax
import jax.numpy as jnp
import numpy as np
from jax.experimental import pallas as pl
from jax.experimental.pallas import tpu as pltpu

R = 2
KNN = 3
SAMP = 3
NEG_BIG = -1e30


def _body(T, M, flat_ref, xs_ref, xsT_ref, wk_ref, b_ref, x_ref, o_ref,
          ycat_s, ny_s):
    j = pl.program_id(1)

    @pl.when(j == 0)
    def _init():
        xsb = xs_ref[0]
        xsT = xsT_ref[0]
        for k in range(KNN):
            ycat_s[k] = jnp.dot(wk_ref[k], xsb,
                                preferred_element_type=jnp.float32)
        ny = jnp.sum(xsT * xsT, axis=1, keepdims=True)
        ny_s[...] = jnp.broadcast_to(ny, ny_s.shape)

    xt = x_ref[0]
    inner = jnp.dot(xsT_ref[0], xt,
                    preferred_element_type=jnp.float32)
    s = ny_s[:, 0:1] - 2.0 * inner
    pos = j * T + jax.lax.broadcasted_iota(jnp.int32, (M, T), 1)
    s = jnp.where(pos == flat_ref[:, 0:1], NEG_BIG, s)
    sj = s[:, None, :]
    sm = s[None, :, :]
    jj = jax.lax.broadcasted_iota(jnp.int32, (M, M, T), 0)
    mm = jax.lax.broadcasted_iota(jnp.int32, (M, M, T), 1)
    beats = (sj < sm) | ((sj == sm) & (jj < mm))
    rank = jnp.sum(beats.astype(jnp.int32), axis=0)
    acc = jnp.dot(ycat_s[0], (rank == 0).astype(jnp.float32),
                  preferred_element_type=jnp.float32)
    acc = acc + jnp.dot(ycat_s[1], (rank == 1).astype(jnp.float32),
                        preferred_element_type=jnp.float32)
    acc = acc + jnp.dot(ycat_s[2], (rank == 2).astype(jnp.float32),
                        preferred_element_type=jnp.float32)
    o_ref[0] = acc + b_ref[:, 0:1]


def kernel(x, W, b):
    B, C, H, Wsp = x.shape
    H2, W2 = H // R, Wsp // R
    C1 = C * R * R
    N = H2 * W2

    x1 = (x.reshape(B, C, H2, R, W2, R)
           .transpose(0, 1, 3, 5, 2, 4)
           .reshape(B, C1, N))
    x_ind = np.round(np.linspace(0.0, H2 - 1.0, SAMP)).astype(np.int64)
    y_ind = np.round(np.linspace(0.0, W2 - 1.0, SAMP)).astype(np.int64)
    xg, yg = np.meshgrid(x_ind, y_ind, indexing="ij")
    xf, yf = xg.flatten(), yg.flatten()
    tok_idx = (xf * W2 + yf).astype(np.int32)
    flat_np = (xf * H2 + yf).astype(np.int32)
    M = int(tok_idx.size)
    xs = x1[:, :, jnp.asarray(tok_idx)]
    xsT = jnp.swapaxes(xs, 1, 2)
    wk = jnp.transpose(W, (2, 0, 1))
    bcol = b.reshape(C1, 1)
    flat_col = jnp.asarray(flat_np).reshape(M, 1)

    T = next(t for t in (3072, 2048, 1024, 512, 256, 128, N) if N % t == 0)

    grid = (B, N // T)
    out = pl.pallas_call(
        functools.partial(_body, T, M),
        grid=grid,
        in_specs=[
            pl.BlockSpec((M, 1), lambda bb, jj_: (0, 0)),
            pl.BlockSpec((1, C1, M), lambda bb, jj_: (bb, 0, 0)),
            pl.BlockSpec((1, M, C1), lambda bb, jj_: (bb, 0, 0)),
            pl.BlockSpec((KNN, C1, C1), lambda bb, jj_: (0, 0, 0)),
            pl.BlockSpec((C1, 1), lambda bb, jj_: (0, 0)),
            pl.BlockSpec((1, C1, T), lambda bb, jj_: (bb, 0, jj_)),
        ],
        out_specs=pl.BlockSpec((1, C1, T), lambda bb, jj_: (bb, 0, jj_)),
        out_shape=jax.ShapeDtypeStruct((B, C1, N), jnp.float32),
        scratch_shapes=[
            pltpu.VMEM((KNN, C1, M), jnp.float32),
            pltpu.VMEM((M, 128), jnp.float32),
        ],
    )(flat_col, xs, xsT, wk, bcol, x1)

    Co = C1 // (R * R)
    x5 = (out.reshape(B, Co, R, R, H2, W2)
             .transpose(0, 1, 4, 2, 5, 3)
             .reshape(B, Co, H2 * R, W2 * R))
    return x5
